# Initial kernel scaffold; baseline (speedup 1.0000x reference)
import jax
import jax.numpy as jnp
from jax.experimental import pallas as pl


def kernel(t_idx, lam):
    raise NotImplementedError("write your pallas kernel here")


# speedup vs baseline: 124.0449x; 124.0449x over previous
import jax
import jax.numpy as jnp
from jax import lax
from jax.experimental import pallas as pl
from jax.experimental.pallas import tpu as pltpu
from jax.experimental.pallas import tpu_sc as plsc

ROWS, COLS = 16384, 200
N = ROWS * COLS
NC, NS, LANES = 2, 16, 16
NW = NC * NS
NPW = N // NW
CHUNK = 12800
NCHUNK = NPW // CHUNK


def _sc_body(idx_hbm, lam_hbm, out_hbm, idx_v, val_v, sem):
    wid = lax.axis_index("s") * NC + lax.axis_index("c")
    base = wid * NPW

    for c in range(NCHUNK):
        off = base + c * CHUNK
        pltpu.sync_copy(idx_hbm.at[pl.ds(off, CHUNK)], idx_v)
        pltpu.async_copy(lam_hbm.at[idx_v], val_v, sem).wait()

        def ew(i, _):
            v = val_v[pl.ds(i * LANES, LANES)]
            val_v[pl.ds(i * LANES, LANES)] = jnp.where(
                v >= 1.0, v, jnp.exp(v - 1.0)
            )
            return 0

        lax.fori_loop(0, CHUNK // LANES, ew, 0, unroll=4)
        pltpu.sync_copy(val_v, out_hbm.at[pl.ds(off, CHUNK)])


def kernel(t_idx, lam):
    idx_flat = t_idx.reshape(N)
    mesh = plsc.VectorSubcoreMesh(core_axis_name="c", subcore_axis_name="s")
    out = pl.kernel(
        _sc_body,
        out_type=jax.ShapeDtypeStruct((N,), jnp.float32),
        mesh=mesh,
        scratch_types=[
            pltpu.VMEM((CHUNK,), jnp.int32),
            pltpu.VMEM((CHUNK,), jnp.float32),
            pltpu.SemaphoreType.DMA,
        ],
    )(idx_flat, lam)
    return out.reshape(ROWS, COLS)


# speedup vs baseline: 139.3111x; 1.1231x over previous
import jax
import jax.numpy as jnp
from jax import lax
from jax.experimental import pallas as pl
from jax.experimental.pallas import tpu as pltpu
from jax.experimental.pallas import tpu_sc as plsc

ROWS, COLS = 16384, 200
N = ROWS * COLS
NC, NS, LANES = 2, 16, 16
NW = NC * NS
NPW = N // NW
CHUNK = 12800
NCHUNK = NPW // CHUNK


def _sc_body(idx_hbm, lam_hbm, out_hbm,
             idx_a, idx_b, val_a, val_b,
             isem_a, isem_b, gsem_a, gsem_b, osem_a, osem_b):
    wid = lax.axis_index("s") * NC + lax.axis_index("c")
    base = wid * NPW

    idx_v = [idx_a, idx_b]
    val_v = [val_a, val_b]
    isem = [isem_a, isem_b]
    gsem = [gsem_a, gsem_b]
    osem = [osem_a, osem_b]

    def start_idx(c, b):
        off = base + c * CHUNK
        pltpu.async_copy(idx_hbm.at[pl.ds(off, CHUNK)], idx_v[b], isem[b])

    start_idx(0, 0)
    pltpu.make_async_copy(idx_hbm.at[pl.ds(base, CHUNK)], idx_v[0], isem[0]).wait()
    pltpu.async_copy(lam_hbm.at[idx_v[0]], val_v[0], gsem[0])

    for c in range(NCHUNK):
        cur = c & 1
        nxt = 1 - cur
        if c + 1 < NCHUNK:
            start_idx(c + 1, nxt)
            pltpu.make_async_copy(
                idx_hbm.at[pl.ds(base, CHUNK)], idx_v[nxt], isem[nxt]
            ).wait()
            if c + 1 >= 2:
                pltpu.make_async_copy(
                    val_v[nxt], out_hbm.at[pl.ds(base, CHUNK)], osem[nxt]
                ).wait()
            pltpu.async_copy(lam_hbm.at[idx_v[nxt]], val_v[nxt], gsem[nxt])

        pltpu.make_async_copy(lam_hbm.at[idx_v[cur]], val_v[cur], gsem[cur]).wait()

        def ew(i, _):
            v = val_v[cur][pl.ds(i * LANES, LANES)]
            val_v[cur][pl.ds(i * LANES, LANES)] = jnp.where(
                v >= 1.0, v, jnp.exp(v - 1.0)
            )
            return 0

        lax.fori_loop(0, CHUNK // LANES, ew, 0, unroll=8)

        off = base + c * CHUNK
        pltpu.async_copy(val_v[cur], out_hbm.at[pl.ds(off, CHUNK)], osem[cur])

    pltpu.make_async_copy(
        val_v[0], out_hbm.at[pl.ds(base, CHUNK)], osem[0]
    ).wait()
    pltpu.make_async_copy(
        val_v[1], out_hbm.at[pl.ds(base, CHUNK)], osem[1]
    ).wait()


def kernel(t_idx, lam):
    idx_flat = t_idx.reshape(N)
    mesh = plsc.VectorSubcoreMesh(core_axis_name="c", subcore_axis_name="s")
    out = pl.kernel(
        _sc_body,
        out_type=jax.ShapeDtypeStruct((N,), jnp.float32),
        mesh=mesh,
        scratch_types=[
            pltpu.VMEM((CHUNK,), jnp.int32),
            pltpu.VMEM((CHUNK,), jnp.int32),
            pltpu.VMEM((CHUNK,), jnp.float32),
            pltpu.VMEM((CHUNK,), jnp.float32),
            pltpu.SemaphoreType.DMA,
            pltpu.SemaphoreType.DMA,
            pltpu.SemaphoreType.DMA,
            pltpu.SemaphoreType.DMA,
            pltpu.SemaphoreType.DMA,
            pltpu.SemaphoreType.DMA,
        ],
    )(idx_flat, lam)
    return out.reshape(ROWS, COLS)


# speedup vs baseline: 217.1607x; 1.5588x over previous
import jax
import jax.numpy as jnp
from jax import lax
from jax.experimental import pallas as pl
from jax.experimental.pallas import tpu as pltpu
from jax.experimental.pallas import tpu_sc as plsc

ROWS, COLS = 16384, 200
TABLE = 1_000_000
N = ROWS * COLS
NC, NS, LANES = 2, 16, 16
NW = NC * NS
NPW = N // NW
CHUNK = 12800
NCHUNK = NPW // CHUNK
TPAD = 1_024_000
TQ = TPAD // NS


def _sc_body(idx_hbm, lam_hbm, out_hbm,
             idx_a, idx_b, val_a, val_b, tab_sh,
             isem_a, isem_b, gsem_a, gsem_b, osem_a, osem_b):
    sid = lax.axis_index("s")
    cid = lax.axis_index("c")
    wid = sid * NC + cid

    toff = sid * TQ
    for j in range(TQ // CHUNK):
        buf = val_a if (j & 1) == 0 else val_b
        o = toff + j * CHUNK
        pltpu.sync_copy(lam_hbm.at[pl.ds(o, CHUNK)], buf)

        def fa(i, _):
            v = buf[pl.ds(i * LANES, LANES)]
            buf[pl.ds(i * LANES, LANES)] = jnp.where(
                v >= 1.0, v, jnp.exp(v - 1.0)
            )
            return 0

        lax.fori_loop(0, CHUNK // LANES, fa, 0, unroll=8)
        pltpu.sync_copy(buf, tab_sh.at[pl.ds(o, CHUNK)])
    plsc.subcore_barrier()

    idx_v = [idx_a, idx_b]
    val_v = [val_a, val_b]
    isem = [isem_a, isem_b]
    gsem = [gsem_a, gsem_b]
    osem = [osem_a, osem_b]
    base = wid * NPW

    def start_idx(c, b):
        off = base + c * CHUNK
        pltpu.async_copy(idx_hbm.at[pl.ds(off, CHUNK)], idx_v[b], isem[b])

    def wait_idx(b):
        pltpu.make_async_copy(
            idx_hbm.at[pl.ds(base, CHUNK)], idx_v[b], isem[b]
        ).wait()

    def wait_out(b):
        pltpu.make_async_copy(
            val_v[b], out_hbm.at[pl.ds(base, CHUNK)], osem[b]
        ).wait()

    def wait_gather(b):
        pltpu.make_async_copy(tab_sh.at[idx_v[b]], val_v[b], gsem[b]).wait()

    start_idx(0, 0)
    wait_idx(0)
    pltpu.async_copy(tab_sh.at[idx_v[0]], val_v[0], gsem[0])

    for c in range(NCHUNK):
        cur = c & 1
        nxt = 1 - cur
        if c + 1 < NCHUNK:
            start_idx(c + 1, nxt)
            wait_idx(nxt)
            if c + 1 >= 2:
                wait_out(nxt)
            pltpu.async_copy(tab_sh.at[idx_v[nxt]], val_v[nxt], gsem[nxt])
        wait_gather(cur)
        off = base + c * CHUNK
        pltpu.async_copy(val_v[cur], out_hbm.at[pl.ds(off, CHUNK)], osem[cur])

    wait_out(0)
    wait_out(1)


def kernel(t_idx, lam):
    lam_p = jnp.pad(lam, (0, TPAD - TABLE))
    idx_flat = t_idx.reshape(N)
    mesh = plsc.VectorSubcoreMesh(core_axis_name="c", subcore_axis_name="s")
    out = pl.kernel(
        _sc_body,
        out_type=jax.ShapeDtypeStruct((N,), jnp.float32),
        mesh=mesh,
        scratch_types=[
            pltpu.VMEM((CHUNK,), jnp.int32),
            pltpu.VMEM((CHUNK,), jnp.int32),
            pltpu.VMEM((CHUNK,), jnp.float32),
            pltpu.VMEM((CHUNK,), jnp.float32),
            pltpu.MemorySpace.VMEM_SHARED((TPAD,), jnp.float32),
            pltpu.SemaphoreType.DMA,
            pltpu.SemaphoreType.DMA,
            pltpu.SemaphoreType.DMA,
            pltpu.SemaphoreType.DMA,
            pltpu.SemaphoreType.DMA,
            pltpu.SemaphoreType.DMA,
        ],
    )(idx_flat, lam_p)
    return out.reshape(ROWS, COLS)
